# baseline (speedup 1.0000x reference)
import functools

import jax
import jax.numpy as jnp
from jax import lax
from jax.experimental import pallas as pl
from jax.experimental.pallas import tpu as pltpu
from jax.experimental.pallas import tpu_sc as plsc

B = 16384
F = 26
V = 100000
D = 32
T = B * F
NW = 32
R = T // NW
CH = 128
NCH = R // CH
NBUF = 4


def _body(x_hbm, tab_hbm, out_hbm, idx_v, rows_v, gsem):
    cid = lax.axis_index("c")
    sid = lax.axis_index("s")
    wid = sid * 2 + cid
    base = wid * R

    pltpu.sync_copy(x_hbm.at[wid], idx_v)

    lanes = lax.iota(jnp.int32, 16)

    def offs_chunk(j, carry):
        def offs_grp(c, carry2):
            pos = base + j * CH + c * 16 + lanes
            grp = idx_v[j, pl.ds(c * 16, 16)]
            idx_v[j, pl.ds(c * 16, 16)] = grp + (pos % F) * V
            return carry2

        return lax.fori_loop(0, CH // 16, offs_grp, carry)

    lax.fori_loop(0, NCH, offs_chunk, 0)

    def start_gather(j, slot):
        pltpu.make_async_copy(
            tab_hbm.at[idx_v.at[j]], rows_v.at[slot], gsem.at[slot]
        ).start()

    def wait_gather(j, slot):
        pltpu.make_async_copy(
            tab_hbm.at[idx_v.at[j]], rows_v.at[slot], gsem.at[slot]
        ).wait()

    for b in range(NBUF):
        start_gather(b, b)

    def outer(jo, carry):
        for b in range(NBUF):
            j = jo * NBUF + b
            wait_gather(j, b)
            pltpu.sync_copy(rows_v.at[b], out_hbm.at[pl.ds(base + j * CH, CH)])
            nj = j + NBUF

            @pl.when(nj < NCH)
            def _():
                start_gather(nj, b)

        return carry

    lax.fori_loop(0, NCH // NBUF, outer, 0)


@jax.jit
def kernel(x, tables):
    xr = x.reshape(NW, NCH, CH)
    tab = tables.reshape(F * V, D)
    mesh = plsc.VectorSubcoreMesh(core_axis_name="c", subcore_axis_name="s")
    out = pl.kernel(
        _body,
        out_type=jax.ShapeDtypeStruct((T, D), jnp.float32),
        mesh=mesh,
        scratch_types=[
            pltpu.VMEM((NCH, CH), jnp.int32),
            pltpu.VMEM((NBUF, CH, D), jnp.float32),
            pltpu.SemaphoreType.DMA((NBUF,)),
        ],
        compiler_params=pltpu.CompilerParams(use_tc_tiling_on_sc=False),
    )(xr, tab)
    return out.reshape(B, F * D)
